# baseline (speedup 1.0000x reference)
import functools

import jax
import jax.numpy as jnp
from jax import lax
from jax.experimental import pallas as pl
from jax.experimental.pallas import tpu as pltpu
from jax.experimental.pallas import tpu_sc as plsc

_NC, _NS = 2, 16
_NW = _NC * _NS
_IW = 128
_G = 10
_L = 16
_BPW = 512


def _sc_gather(tt, table):
    S = tt.shape[0]
    D = table.shape[1]
    rpw = _BPW * S
    n_chunks = rpw // (_G * _IW)
    mesh = plsc.VectorSubcoreMesh(
        core_axis_name="c", subcore_axis_name="s",
        num_cores=_NC, num_subcores=_NS)

    @functools.partial(
        pl.kernel,
        out_type=jax.ShapeDtypeStruct((_NW * rpw, D), jnp.float32),
        mesh=mesh,
        scratch_types=[
            pltpu.VMEM((S, _IW), jnp.int32),
            pltpu.VMEM((rpw,), jnp.int32),
            pltpu.VMEM((2, _G * _IW, D), jnp.float32),
            pltpu.SemaphoreType.DMA,
            pltpu.SemaphoreType.DMA,
            pltpu.SemaphoreType.DMA,
        ],
        compiler_params=pltpu.CompilerParams(
            use_tc_tiling_on_sc=False, needs_layout_passes=False),
    )
    def k(tt_hbm, tab_hbm, out_hbm, idxL_v, idxT_v, rows_v, gsem, wsem0, wsem1):
        w = lax.axis_index("s") * _NC + lax.axis_index("c")
        b0 = w * _BPW
        row0 = w * rpw
        wsems = (wsem0, wsem1)
        lane_s = lax.iota(jnp.int32, _L) * S

        for c in range(_BPW // _IW):
            pltpu.sync_copy(tt_hbm.at[:, pl.ds(b0 + c * _IW, _IW)], idxL_v)

            @pl.loop(0, S)
            def _s(s, c=c):
                for g in range(_IW // _L):
                    v = idxL_v[s, pl.ds(g * _L, _L)]
                    tgt = lane_s + (c * _IW * S + g * _L * S) + s
                    plsc.store_scatter(idxT_v, [tgt], v)

        def do_chunk(cn, b, wait_writeback):
            base = cn * _G * _IW
            if wait_writeback:
                pltpu.make_async_copy(
                    rows_v.at[b], out_hbm.at[pl.ds(row0 + base, _G * _IW)],
                    wsems[b]).wait()
            copies = [
                pltpu.async_copy(
                    tab_hbm.at[idxT_v.at[pl.ds(base + j * _IW, _IW)]],
                    rows_v.at[b, pl.ds(j * _IW, _IW)], gsem)
                for j in range(_G)
            ]
            for c in copies:
                c.wait()
            pltpu.async_copy(
                rows_v.at[b], out_hbm.at[pl.ds(row0 + base, _G * _IW)],
                wsems[b])

        do_chunk(0, 0, False)
        do_chunk(1, 1, False)

        @pl.loop(2, n_chunks, step=2)
        def _pair(cn):
            do_chunk(cn, 0, True)
            do_chunk(cn + 1, 1, True)

        for b in range(2):
            base = (n_chunks - 2 + b) * _G * _IW
            pltpu.make_async_copy(
                rows_v.at[b], out_hbm.at[pl.ds(row0 + base, _G * _IW)],
                wsems[b]).wait()

    return k(tt, table)


def kernel(token_ids, E):
    B0, B1 = token_ids.shape
    D = E.shape[1]
    tt = token_ids.T.astype(jnp.int32)
    out = _sc_gather(tt, E)
    return out.reshape(B0, B1, D)


# speedup vs baseline: 1.2299x; 1.2299x over previous
import functools

import jax
import jax.numpy as jnp
from jax import lax
from jax.experimental import pallas as pl
from jax.experimental.pallas import tpu as pltpu
from jax.experimental.pallas import tpu_sc as plsc

_NC, _NS = 2, 16
_NW = _NC * _NS
_IW = 128
_GPB = 4
_BLK = _GPB * _IW
_L = 16
_DP = 33


def _sc_gather(tt3, table33):
    S = tt3.shape[0]
    B = _NW * _BLK
    D = 32
    mesh = plsc.VectorSubcoreMesh(
        core_axis_name="c", subcore_axis_name="s",
        num_cores=_NC, num_subcores=_NS)

    @functools.partial(
        pl.kernel,
        out_type=jax.ShapeDtypeStruct((S, D, B), jnp.float32),
        mesh=mesh,
        scratch_types=[
            pltpu.VMEM((2, _GPB, _IW), jnp.int32),
            pltpu.VMEM((2, _BLK, _DP), jnp.float32),
            pltpu.VMEM((2, D, _BLK), jnp.float32),
            pltpu.SemaphoreType.DMA,
            pltpu.SemaphoreType.DMA,
            pltpu.SemaphoreType.DMA,
        ],
        compiler_params=pltpu.CompilerParams(
            use_tc_tiling_on_sc=False, needs_layout_passes=False),
    )
    def k(idx_hbm, tab_hbm, out_hbm, idx_v, rows_v, rowsT_v, gsem, wsem0, wsem1):
        w = lax.axis_index("s") * _NC + lax.axis_index("c")
        b0 = w * _BLK
        wsems = (wsem0, wsem1)
        lane = lax.iota(jnp.int32, _L)

        def fire(s, b):
            pltpu.sync_copy(idx_hbm.at[s, pl.ds(w * _GPB, _GPB)], idx_v.at[b])
            for j in range(_GPB):
                pltpu.async_copy(
                    tab_hbm.at[idx_v.at[b].at[j]],
                    rows_v.at[b, pl.ds(j * _IW, _IW)], gsem)

        def dtw(s, b, wait_writeback):
            for j in range(_GPB):
                pltpu.make_async_copy(
                    tab_hbm.at[idx_v.at[b].at[j]],
                    rows_v.at[b, pl.ds(j * _IW, _IW)], gsem).wait()
            if wait_writeback:
                pltpu.make_async_copy(
                    rowsT_v.at[b], out_hbm.at[s, :, pl.ds(b0, _BLK)],
                    wsems[b]).wait()

            @pl.loop(0, _BLK // _L)
            def _tg(g):
                row_ids = g * _L + lane
                for d in range(D):
                    v = plsc.load_gather(
                        rows_v.at[b],
                        [row_ids, jnp.full((_L,), d, jnp.int32)])
                    rowsT_v[b, d, pl.ds(g * _L, _L)] = v

            pltpu.async_copy(
                rowsT_v.at[b], out_hbm.at[s, :, pl.ds(b0, _BLK)], wsems[b])

        fire(0, 0)
        fire(1, 1)
        dtw(0, 0, False)
        fire(2, 0)
        dtw(1, 1, False)
        fire(3, 1)
        dtw(2, 0, True)

        @pl.loop(4, S, step=2)
        def _pair(s):
            fire(s, 0)
            dtw(s - 1, 1, True)
            fire(s + 1, 1)
            dtw(s, 0, True)

        dtw(S - 1, 1, True)
        for b in range(2):
            pltpu.make_async_copy(
                rowsT_v.at[b], out_hbm.at[S - 2 + b, :, pl.ds(b0, _BLK)],
                wsems[b]).wait()

    return k(tt3, table33)


def kernel(token_ids, E):
    B0, B1 = token_ids.shape
    D = E.shape[1]
    tt3 = token_ids.T.reshape(B1, B0 // _IW, _IW).astype(jnp.int32)
    table33 = jnp.pad(E, ((0, 0), (0, _DP - D)))
    out = _sc_gather(tt3, table33)
    return jnp.transpose(out, (2, 0, 1))


# speedup vs baseline: 1.9525x; 1.5875x over previous
import functools

import jax
import jax.numpy as jnp
from jax import lax
from jax.experimental import pallas as pl
from jax.experimental.pallas import tpu as pltpu
from jax.experimental.pallas import tpu_sc as plsc

_NC, _NS = 2, 16
_NW = _NC * _NS
_IW = 128
_GPB = 4
_BLK = _GPB * _IW
_L = 16


def _sc_gather(tt3, table):
    S = tt3.shape[0]
    B = _NW * _BLK
    D = 32
    mesh = plsc.VectorSubcoreMesh(
        core_axis_name="c", subcore_axis_name="s",
        num_cores=_NC, num_subcores=_NS)

    @functools.partial(
        pl.kernel,
        out_type=jax.ShapeDtypeStruct((S, D, B), jnp.float32),
        mesh=mesh,
        scratch_types=[
            pltpu.VMEM((2, _GPB, _IW), jnp.int32),
            pltpu.VMEM((2, _BLK, D), jnp.float32),
            pltpu.VMEM((2, D, _BLK), jnp.float32),
            pltpu.SemaphoreType.DMA,
            pltpu.SemaphoreType.DMA,
            pltpu.SemaphoreType.DMA,
            pltpu.SemaphoreType.DMA,
        ],
        compiler_params=pltpu.CompilerParams(
            use_tc_tiling_on_sc=False, needs_layout_passes=False),
    )
    def k(idx_hbm, tab_hbm, out_hbm, idx_v, rows_v, rowsT_v, gsem0, gsem1, wsem0, wsem1):
        w = lax.axis_index("s") * _NC + lax.axis_index("c")
        b0 = w * _BLK
        gsems = (gsem0, gsem1)
        wsems = (wsem0, wsem1)
        lane = lax.iota(jnp.int32, _L)

        def fire(s, b):
            pltpu.sync_copy(idx_hbm.at[s, pl.ds(w * _GPB, _GPB)], idx_v.at[b])
            for j in range(_GPB):
                pltpu.async_copy(
                    tab_hbm.at[idx_v.at[b].at[j]],
                    rows_v.at[b, pl.ds(j * _IW, _IW)], gsems[b])

        def dtw(s, b, wait_writeback):
            for j in range(_GPB):
                pltpu.make_async_copy(
                    tab_hbm.at[idx_v.at[b].at[j]],
                    rows_v.at[b, pl.ds(j * _IW, _IW)], gsems[b]).wait()
            if wait_writeback:
                pltpu.make_async_copy(
                    rowsT_v.at[b], out_hbm.at[s, :, pl.ds(b0, _BLK)],
                    wsems[b]).wait()

            @pl.loop(0, _BLK // _L)
            def _tg(g):
                row_ids = g * _L + lane
                for d in range(D):
                    col_d = jnp.bitwise_and(lane + d, D - 1)
                    v = plsc.load_gather(rows_v.at[b], [row_ids, col_d])
                    plsc.store_scatter(rowsT_v.at[b], [col_d, row_ids], v)

            pltpu.async_copy(
                rowsT_v.at[b], out_hbm.at[s, :, pl.ds(b0, _BLK)], wsems[b])

        fire(0, 0)
        fire(1, 1)
        dtw(0, 0, False)
        fire(2, 0)
        dtw(1, 1, False)
        fire(3, 1)
        dtw(2, 0, True)

        @pl.loop(4, S, step=2)
        def _pair(s):
            fire(s, 0)
            dtw(s - 1, 1, True)
            fire(s + 1, 1)
            dtw(s, 0, True)

        dtw(S - 1, 1, True)
        for b in range(2):
            pltpu.make_async_copy(
                rowsT_v.at[b], out_hbm.at[S - 2 + b, :, pl.ds(b0, _BLK)],
                wsems[b]).wait()

    return k(tt3, table)


def kernel(token_ids, E):
    B0, B1 = token_ids.shape
    D = E.shape[1]
    tt3 = token_ids.T.reshape(B1, B0 // _IW, _IW).astype(jnp.int32)
    out = _sc_gather(tt3, E)
    return jnp.transpose(out, (2, 0, 1))


# speedup vs baseline: 2.0048x; 1.0268x over previous
import functools

import jax
import jax.numpy as jnp
from jax import lax
from jax.experimental import pallas as pl
from jax.experimental.pallas import tpu as pltpu
from jax.experimental.pallas import tpu_sc as plsc

_NC, _NS = 2, 16
_NW = _NC * _NS
_IW = 128
_GPB = 4
_BLK = _GPB * _IW
_L = 16


def _sc_gather(idx_flat, table, S):
    B = _NW * _BLK
    D = 32
    rpw = _BLK * S
    mesh = plsc.VectorSubcoreMesh(
        core_axis_name="c", subcore_axis_name="s",
        num_cores=_NC, num_subcores=_NS)

    @functools.partial(
        pl.kernel,
        out_type=jax.ShapeDtypeStruct((S, D, B), jnp.float32),
        mesh=mesh,
        scratch_types=[
            pltpu.VMEM((_BLK * S,), jnp.int32),
            pltpu.VMEM((S * _BLK,), jnp.int32),
            pltpu.VMEM((2, _BLK, D), jnp.float32),
            pltpu.VMEM((2, D, _BLK), jnp.float32),
            pltpu.SemaphoreType.DMA,
            pltpu.SemaphoreType.DMA,
            pltpu.SemaphoreType.DMA,
            pltpu.SemaphoreType.DMA,
        ],
        compiler_params=pltpu.CompilerParams(
            use_tc_tiling_on_sc=False, needs_layout_passes=False),
    )
    def k(idx_hbm, tab_hbm, out_hbm, idxB_v, idxS_v, rows_v, rowsT_v,
          gsem0, gsem1, wsem0, wsem1):
        w = lax.axis_index("s") * _NC + lax.axis_index("c")
        b0 = w * _BLK
        gsems = (gsem0, gsem1)
        wsems = (wsem0, wsem1)
        lane = lax.iota(jnp.int32, _L)

        pltpu.sync_copy(idx_hbm.at[pl.ds(w * rpw, rpw)], idxB_v)

        lane_s = lane * S

        @pl.loop(0, _BLK // _L)
        def _ti(g):
            row_ids = g * _L + lane
            for s in range(S):
                v = plsc.load_gather(idxB_v, [lane_s + (g * _L * S + s)])
                plsc.store_scatter(idxS_v, [row_ids + s * _BLK], v)

        def fire(s, b):
            for j in range(_GPB):
                pltpu.async_copy(
                    tab_hbm.at[idxS_v.at[pl.ds(s * _BLK + j * _IW, _IW)]],
                    rows_v.at[b, pl.ds(j * _IW, _IW)], gsems[b])

        def dtw(s, b, wait_writeback):
            for j in range(_GPB):
                pltpu.make_async_copy(
                    tab_hbm.at[idxS_v.at[pl.ds(s * _BLK + j * _IW, _IW)]],
                    rows_v.at[b, pl.ds(j * _IW, _IW)], gsems[b]).wait()
            if wait_writeback:
                pltpu.make_async_copy(
                    rowsT_v.at[b], out_hbm.at[s, :, pl.ds(b0, _BLK)],
                    wsems[b]).wait()

            @pl.loop(0, _BLK // _L)
            def _tg(g):
                row_ids = g * _L + lane
                for d in range(D):
                    col_d = jnp.bitwise_and(lane + d, D - 1)
                    v = plsc.load_gather(rows_v.at[b], [row_ids, col_d])
                    plsc.store_scatter(rowsT_v.at[b], [col_d, row_ids], v)

            pltpu.async_copy(
                rowsT_v.at[b], out_hbm.at[s, :, pl.ds(b0, _BLK)], wsems[b])

        fire(0, 0)
        fire(1, 1)
        dtw(0, 0, False)
        fire(2, 0)
        dtw(1, 1, False)
        fire(3, 1)
        dtw(2, 0, True)

        @pl.loop(4, S, step=2)
        def _pair(s):
            fire(s, 0)
            dtw(s - 1, 1, True)
            fire(s + 1, 1)
            dtw(s, 0, True)

        dtw(S - 1, 1, True)
        for b in range(2):
            pltpu.make_async_copy(
                rowsT_v.at[b], out_hbm.at[S - 2 + b, :, pl.ds(b0, _BLK)],
                wsems[b]).wait()

    return k(idx_flat, table)


def kernel(token_ids, E):
    B0, B1 = token_ids.shape
    idx_flat = token_ids.reshape(-1).astype(jnp.int32)
    out = _sc_gather(idx_flat, E, B1)
    return jnp.transpose(out, (2, 0, 1))
